# Initial kernel scaffold; baseline (speedup 1.0000x reference)
import functools

import jax
import jax.numpy as jnp
from jax import lax
from jax.experimental import pallas as pl
from jax.experimental.pallas import tpu as pltpu
from jax.experimental.pallas import tpu_sc as plsc

N = 10000
E = 320000
IN_F = 129
OUT_F = 128
D = 144
CH = 80
NC = 2
NS = 16
NW = NC * NS
EPT = E // NW
CPT = EPT // CH
RPT = N // NS
BLK = 1000


def _norm_pts(f, h):
    zero = jnp.all(f == 0.0, axis=-1, keepdims=True)
    f1 = jnp.where(zero, 1.0, f)
    nrm = jnp.sqrt(jnp.sum(f1 * f1, axis=-1, keepdims=True))
    nf = f1 / jnp.maximum(nrm, 1e-8)
    sg = jnp.sign(h)
    sg = jnp.where(sg == 0.0, 1.0, sg)
    return nf * sg, h * sg


def _pre_body(xf_ref, xh_ref, ws_ref, wn_ref, p1f_ref, p1h_ref, tab_ref):
    xf, xh = _norm_pts(xf_ref[...], xh_ref[...])

    def transform(w):
        t = lax.dot_general(xf, w, (((1,), (1,)), ((), ())),
                            preferred_element_type=jnp.float32,
                            precision=lax.Precision.HIGHEST)
        nrm = jnp.sqrt(jnp.sum(t * t, axis=-1, keepdims=True))
        t = t / jnp.maximum(nrm, 1e-8)
        return _norm_pts(t, xh)

    sf, sh = transform(ws_ref[...])
    nf, nh = transform(wn_ref[...])
    p1f_ref[...] = sf
    p1h_ref[...] = sh
    ones = jnp.ones((BLK, 1), jnp.float32)
    pad = jnp.zeros((BLK, D - OUT_F - 2), jnp.float32)
    tab_ref[...] = jnp.concatenate([nf, nh, ones, pad], axis=-1)


def _post_body(s0_ref, s1_ref, p1f_ref, p1h_ref, out_ref):
    s = s0_ref[...] + s1_ref[...]
    cnt = jnp.maximum(s[:, OUT_F + 1:OUT_F + 2], 1.0)
    feats = s[:, :OUT_F] / cnt
    hom = 1.0 + s[:, OUT_F:OUT_F + 1]
    of, oh = _norm_pts(feats, hom)
    p2f, p2h = _norm_pts(of, oh)
    p1f, p1h = _norm_pts(p1f_ref[...], p1h_ref[...])
    t = jnp.clip(jnp.float32(0.5) / (jnp.float32(1.0) - jnp.float32(0.5)
                                     + jnp.float32(1e-8)), 1e-8, 1e8)
    den = jnp.maximum(t + 1.0, 1e-8)
    af = (p1f * t + p2f) / den
    ah = (p1h * t + p2h) / den
    cf, chh = _norm_pts(af, ah)
    cf, chh = _norm_pts(cf, chh)
    out_ref[...] = jnp.concatenate([cf, chh], axis=-1)


def _sc_body(tab_hbm, row_hbm, col_hbm, zero_hbm, out_hbm,
             colv, rowv, gbuf, acc, sem):
    c = lax.axis_index("c")
    s = lax.axis_index("s")
    pltpu.sync_copy(zero_hbm.at[pl.ds(s * RPT, RPT)],
                    acc.at[pl.ds(s * RPT, RPT)])
    tbase = (c * NS + s) * CPT
    pltpu.sync_copy(col_hbm.at[pl.ds(tbase, CPT)], colv)
    pltpu.sync_copy(row_hbm.at[pl.ds(tbase, CPT)], rowv)
    plsc.subcore_barrier()

    def step(j, carry):
        pltpu.async_copy(tab_hbm.at[colv.at[j]], gbuf, sem).wait()
        pltpu.sync_copy(gbuf, acc.at[rowv.at[j]], add=True)
        return carry

    lax.fori_loop(0, CPT, step, 0)
    plsc.subcore_barrier()
    pltpu.sync_copy(acc.at[pl.ds(s * RPT, RPT)],
                    out_hbm.at[c, pl.ds(s * RPT, RPT)])


_sc_scatter = functools.partial(
    pl.kernel,
    out_type=jax.ShapeDtypeStruct((NC, N, D), jnp.float32),
    mesh=plsc.VectorSubcoreMesh(core_axis_name="c", subcore_axis_name="s",
                                num_cores=NC, num_subcores=NS),
    scratch_types=[
        pltpu.VMEM((CPT, CH), jnp.int32),
        pltpu.VMEM((CPT, CH), jnp.int32),
        pltpu.VMEM((CH, D), jnp.float32),
        pltpu.VMEM_SHARED((N, D), jnp.float32),
        pltpu.SemaphoreType.DMA,
    ],
)(_sc_body)


def kernel(x, edge_index, W_self, W_neigh):
    xf = x[:, :IN_F - 1]
    xh = x[:, IN_F - 1:]
    grid = N // BLK
    p1f, p1h, tab = pl.pallas_call(
        _pre_body,
        grid=(grid,),
        in_specs=[
            pl.BlockSpec((BLK, IN_F - 1), lambda i: (i, 0)),
            pl.BlockSpec((BLK, 1), lambda i: (i, 0)),
            pl.BlockSpec((OUT_F, IN_F - 1), lambda i: (0, 0)),
            pl.BlockSpec((OUT_F, IN_F - 1), lambda i: (0, 0)),
        ],
        out_specs=[
            pl.BlockSpec((BLK, OUT_F), lambda i: (i, 0)),
            pl.BlockSpec((BLK, 1), lambda i: (i, 0)),
            pl.BlockSpec((BLK, D), lambda i: (i, 0)),
        ],
        out_shape=[
            jax.ShapeDtypeStruct((N, OUT_F), jnp.float32),
            jax.ShapeDtypeStruct((N, 1), jnp.float32),
            jax.ShapeDtypeStruct((N, D), jnp.float32),
        ],
    )(xf, xh, W_self, W_neigh)

    row2 = edge_index[0].reshape(E // CH, CH)
    col2 = edge_index[1].reshape(E // CH, CH)
    zeros = jnp.zeros((N, D), jnp.float32)
    partial = _sc_scatter(tab, row2, col2, zeros)

    out = pl.pallas_call(
        _post_body,
        grid=(grid,),
        in_specs=[
            pl.BlockSpec((BLK, D), lambda i: (i, 0)),
            pl.BlockSpec((BLK, D), lambda i: (i, 0)),
            pl.BlockSpec((BLK, OUT_F), lambda i: (i, 0)),
            pl.BlockSpec((BLK, 1), lambda i: (i, 0)),
        ],
        out_specs=pl.BlockSpec((BLK, IN_F), lambda i: (i, 0)),
        out_shape=jax.ShapeDtypeStruct((N, IN_F), jnp.float32),
    )(partial[0], partial[1], p1f, p1h)
    return out


# speedup vs baseline: 3.3944x; 3.3944x over previous
import functools

import jax
import jax.numpy as jnp
from jax import lax
from jax.experimental import pallas as pl
from jax.experimental.pallas import tpu as pltpu
from jax.experimental.pallas import tpu_sc as plsc

N = 10000
E = 320000
IN_F = 129
OUT_F = 128
CH = 80
NC = 2
NS = 16
NW = NC * NS
CPN = 256
EPAD = NS * CPN * CH
G = 32
NGRP = CPN // G
OWN = 5120
ACC_R = OWN + 128
WPT = ACC_R // NS
WBT = OWN // NS
NP = 10240
BLK = 1000
L = 16


def _norm_pts(f, h):
    zero = jnp.all(f == 0.0, axis=-1, keepdims=True)
    f1 = jnp.where(zero, 1.0, f)
    nrm = jnp.sqrt(jnp.sum(f1 * f1, axis=-1, keepdims=True))
    nf = f1 / jnp.maximum(nrm, 1e-8)
    sg = jnp.sign(h)
    sg = jnp.where(sg == 0.0, 1.0, sg)
    return nf * sg, h * sg


def _pre_body(xf_ref, xh_ref, ws_ref, wn_ref, p1f_ref, p1h_ref, tab_ref):
    xf, xh = _norm_pts(xf_ref[...], xh_ref[...])

    def transform(w):
        t = lax.dot_general(xf, w, (((1,), (1,)), ((), ())),
                            preferred_element_type=jnp.float32,
                            precision=lax.Precision.HIGHEST)
        nrm = jnp.sqrt(jnp.sum(t * t, axis=-1, keepdims=True))
        t = t / jnp.maximum(nrm, 1e-8)
        return _norm_pts(t, xh)

    sf, sh = transform(ws_ref[...])
    nf, _ = transform(wn_ref[...])
    p1f_ref[...] = sf
    p1h_ref[...] = sh
    tab_ref[...] = nf


def _post_body(fs_ref, hp_ref, p1f_ref, p1h_ref, out_ref):
    fsum = fs_ref[...]
    homsum = jnp.sum(hp_ref[...], axis=-1, keepdims=True)
    of, oh = _norm_pts(fsum, 1.0 + homsum)
    p2f, p2h = _norm_pts(of, oh)
    p1f, p1h = _norm_pts(p1f_ref[...], p1h_ref[...])
    t = jnp.clip(jnp.float32(0.5) / (jnp.float32(1.0) - jnp.float32(0.5)
                                     + jnp.float32(1e-8)), 1e-8, 1e8)
    den = jnp.maximum(t + 1.0, 1e-8)
    af = (p1f * t + p2f) / den
    ah = (p1h * t + p2h) / den
    cf, chh = _norm_pts(af, ah)
    cf, chh = _norm_pts(cf, chh)
    out_ref[...] = jnp.concatenate([cf, chh], axis=-1)


def _sc_body(tab_hbm, hom_hbm, row_hbm, col_hbm, zero_hbm,
             feat_out, hom_out,
             colv, rowv, gbuf, idxb, homtab, homacc, acc, sem):
    c = lax.axis_index("c")
    s = lax.axis_index("s")
    pltpu.sync_copy(zero_hbm, acc.at[pl.ds(s * WPT, WPT)])
    pltpu.sync_copy(hom_hbm, homtab)
    pltpu.sync_copy(col_hbm.at[s, pl.ds(0, G)], colv.at[0])
    pltpu.sync_copy(row_hbm.at[s, pl.ds(0, G)], rowv.at[0])

    def zstep(j, carry):
        homacc[pl.ds(j * L, L)] = jnp.zeros((L,), jnp.float32)
        return carry

    lax.fori_loop(0, NP // L, zstep, 0)
    plsc.subcore_barrier()

    base = c * OWN

    pltpu.async_copy(tab_hbm.at[colv.at[0, 0]], gbuf.at[0], sem)

    for g in range(NGRP):
        p = g % 2
        if g + 1 < NGRP:
            pltpu.sync_copy(col_hbm.at[s, pl.ds((g + 1) * G, G)],
                            colv.at[(g + 1) % 2])
            pltpu.sync_copy(row_hbm.at[s, pl.ds((g + 1) * G, G)],
                            rowv.at[(g + 1) % 2])

        def step(j2, carry, g=g, p=p):
            @pl.when(j2 + 1 < G)
            def _():
                pltpu.async_copy(tab_hbm.at[colv.at[p, j2 + 1]],
                                 gbuf.at[(j2 + 1) % 2], sem)

            if g + 1 < NGRP:
                @pl.when(j2 + 1 == G)
                def _():
                    pltpu.async_copy(tab_hbm.at[colv.at[(g + 1) % 2, 0]],
                                     gbuf.at[(j2 + 1) % 2], sem)

            def cidx(k, carry2):
                rows = rowv[p, j2, pl.ds(k * L, L)]
                rel = rows - base
                ok = (rel >= 0) & (rel < OWN)
                idxb[pl.ds(k * L, L)] = jnp.where(ok, rel, OWN)
                return carry2

            lax.fori_loop(0, CH // L, cidx, 0)
            pltpu.make_async_copy(tab_hbm.at[colv.at[p, j2]],
                                  gbuf.at[j2 % 2], sem).wait()
            pltpu.sync_copy(gbuf.at[j2 % 2], acc.at[idxb], add=True)
            return carry

        lax.fori_loop(0, G, step, 0)

        @pl.when(c == g // (NGRP // NC))
        def _(p=p):
            def hstep(j2, carry):
                def inner(k, carry2):
                    cols = colv[p, j2, pl.ds(k * L, L)]
                    rows = rowv[p, j2, pl.ds(k * L, L)]
                    vals = plsc.load_gather(homtab, [cols])
                    plsc.addupdate_scatter(homacc, [rows], vals)
                    return carry2

                lax.fori_loop(0, CH // L, inner, 0)
                return carry

            lax.fori_loop(0, G, hstep, 0)

    pltpu.sync_copy(homacc, hom_out.at[c, s])
    plsc.subcore_barrier()
    pltpu.sync_copy(acc.at[pl.ds(s * WBT, WBT)],
                    feat_out.at[c, pl.ds(s * WBT, WBT)])


@functools.cache
def _sc_scatter():
    return pl.kernel(
        _sc_body,
        out_type=(
            jax.ShapeDtypeStruct((NC, OWN, OUT_F), jnp.float32),
            jax.ShapeDtypeStruct((NC, NS, NP), jnp.float32),
        ),
        mesh=plsc.VectorSubcoreMesh(core_axis_name="c", subcore_axis_name="s",
                                    num_cores=NC, num_subcores=NS),
        compiler_params=pltpu.CompilerParams(needs_layout_passes=False),
        scratch_types=[
            pltpu.VMEM((2, G, CH), jnp.int32),
            pltpu.VMEM((2, G, CH), jnp.int32),
            pltpu.VMEM((2, CH, OUT_F), jnp.float32),
            pltpu.VMEM((CH,), jnp.int32),
            pltpu.VMEM((N,), jnp.float32),
            pltpu.VMEM((NP,), jnp.float32),
            pltpu.VMEM_SHARED((ACC_R, OUT_F), jnp.float32),
            pltpu.SemaphoreType.DMA,
        ],
    )


def kernel(x, edge_index, W_self, W_neigh):
    xf = x[:, :IN_F - 1]
    xh = x[:, IN_F - 1:]
    grid = N // BLK
    p1f, p1h, tab = pl.pallas_call(
        _pre_body,
        grid=(grid,),
        in_specs=[
            pl.BlockSpec((BLK, IN_F - 1), lambda i: (i, 0)),
            pl.BlockSpec((BLK, 1), lambda i: (i, 0)),
            pl.BlockSpec((OUT_F, IN_F - 1), lambda i: (0, 0)),
            pl.BlockSpec((OUT_F, IN_F - 1), lambda i: (0, 0)),
        ],
        out_specs=[
            pl.BlockSpec((BLK, OUT_F), lambda i: (i, 0)),
            pl.BlockSpec((BLK, 1), lambda i: (i, 0)),
            pl.BlockSpec((BLK, OUT_F), lambda i: (i, 0)),
        ],
        out_shape=[
            jax.ShapeDtypeStruct((N, OUT_F), jnp.float32),
            jax.ShapeDtypeStruct((N, 1), jnp.float32),
            jax.ShapeDtypeStruct((N, OUT_F), jnp.float32),
        ],
    )(xf, xh, W_self, W_neigh)

    rpad = jnp.full((EPAD - E,), NP - 1, jnp.int32)
    cpad = jnp.zeros((EPAD - E,), jnp.int32)
    row3 = jnp.concatenate([edge_index[0], rpad]).reshape(NS, CPN, CH)
    col3 = jnp.concatenate([edge_index[1], cpad]).reshape(NS, CPN, CH)
    zeros = jnp.zeros((WPT, OUT_F), jnp.float32)
    feat_part, hom_part = _sc_scatter()(tab, p1h.reshape(N), row3, col3, zeros)
    feat_full = feat_part.reshape(NC * OWN, OUT_F)
    hom_part_t = hom_part.reshape(NW, NP).T

    out = pl.pallas_call(
        _post_body,
        grid=(grid,),
        in_specs=[
            pl.BlockSpec((BLK, OUT_F), lambda i: (i, 0)),
            pl.BlockSpec((BLK, NW), lambda i: (i, 0)),
            pl.BlockSpec((BLK, OUT_F), lambda i: (i, 0)),
            pl.BlockSpec((BLK, 1), lambda i: (i, 0)),
        ],
        out_specs=pl.BlockSpec((BLK, IN_F), lambda i: (i, 0)),
        out_shape=jax.ShapeDtypeStruct((N, IN_F), jnp.float32),
    )(feat_full, hom_part_t, p1f, p1h)
    return out
